# baseline (speedup 1.0000x reference)
import functools
import math

import jax
import jax.numpy as jnp
from jax import lax
from jax.experimental import pallas as pl
from jax.experimental.pallas import tpu as pltpu
from jax.experimental.pallas import tpu_sc as plsc

HIDDEN = 64
HIST = 20
SCALE = math.sqrt(HIDDEN)
LANES = 16
NCORES = 2
NSUB = 16
NW = NCORES * NSUB

CB = 8
WIDE = 128

VOCAB_ROWS = 1000000
CPY = 160
N_CPY = VOCAB_ROWS // CPY


def _make_pack_kernel():
  per_w = N_CPY // NW
  leftover = N_CPY - per_w * NW

  mesh = plsc.VectorSubcoreMesh(core_axis_name="c", subcore_axis_name="s")

  @functools.partial(
      pl.kernel,
      out_type=jax.ShapeDtypeStruct((VOCAB_ROWS // 2, WIDE), jnp.float32),
      mesh=mesh,
      scratch_types=[
          pltpu.VMEM((CPY, HIDDEN), jnp.float32),
          pltpu.VMEM((CPY, HIDDEN), jnp.float32),
          pltpu.VMEM((CPY // 2, WIDE), jnp.float32),
          pltpu.VMEM((CPY // 2, WIDE), jnp.float32),
          pltpu.SemaphoreType.DMA,
          pltpu.SemaphoreType.DMA,
          pltpu.SemaphoreType.DMA,
          pltpu.SemaphoreType.DMA,
      ],
  )
  def pack_kernel(table_hbm, packed_hbm, va0, va1, vb0, vb1,
                  li0, li1, lo0, lo1):
    vas = (va0, va1)
    vbs = (vb0, vb1)
    lis = (li0, li1)
    los = (lo0, lo1)

    wid = lax.axis_index("s") * NCORES + lax.axis_index("c")
    base = wid * CPY * per_w

    def load(c, b, base_rows):
      off = pl.multiple_of(base_rows + c * CPY, 8)
      return pltpu.make_async_copy(
          table_hbm.at[pl.ds(off, CPY)], vas[b], lis[b])

    def store(c, b, base_rows):
      off = pl.multiple_of(base_rows // 2 + c * (CPY // 2), 8)
      return pltpu.make_async_copy(
          vbs[b], packed_hbm.at[pl.ds(off, CPY // 2)], los[b])

    def pack(b):
      va = vas[b]
      vb = vbs[b]

      def body(k, carry):
        for par in (0, 1):
          for j in range(HIDDEN // LANES):
            vb[k, pl.ds(par * HIDDEN + j * LANES, LANES)] = (
                va[2 * k + par, pl.ds(j * LANES, LANES)])
        return carry

      lax.fori_loop(0, CPY // 2, body, 0)

    def chunk_step(c, b):
      load(c, b, base).wait()

      @pl.when(c >= 2)
      def _():
        store(c - 2, b, base).wait()

      pack(b)

      @pl.when(c + 2 < per_w)
      def _():
        load(c + 2, b, base).start()

      store(c, b, base).start()

    def loop_body(c2, carry):
      for b in (0, 1):
        chunk_step(c2 * 2 + b, b)
      return carry

    load(0, 0, base).start()
    load(1, 1, base).start()
    lax.fori_loop(0, per_w // 2, loop_body, 0)
    if per_w % 2:
      c = per_w - 1
      load(c, 0, base).wait()
      store(c - 2, 0, base).wait()
      pack(0)
      store(c, 0, base).start()
    store(per_w - 2, (per_w - 2) % 2, base).wait()
    store(per_w - 1, (per_w - 1) % 2, base).wait()

    @pl.when(wid < leftover)
    def _():
      t0 = NW * per_w * CPY + wid * CPY
      load(0, 0, t0).start()
      load(0, 0, t0).wait()
      pack(0)
      store(0, 0, t0).start()
      store(0, 0, t0).wait()

  return pack_kernel


def _make_gather_kernel(batch: int):
  assert batch % (NW * CB) == 0
  items_per_w = batch // NW
  nchunks = items_per_w // CB
  rpc = CB * HIST
  groups = rpc // LANES
  vpr = HIDDEN // LANES

  mesh = plsc.VectorSubcoreMesh(core_axis_name="c", subcore_axis_name="s")

  @functools.partial(
      pl.kernel,
      out_type=jax.ShapeDtypeStruct((batch, HIST, HIDDEN), jnp.float32),
      mesh=mesh,
      scratch_types=[
          pltpu.VMEM((rpc,), jnp.int32),
          pltpu.VMEM((rpc,), jnp.int32),
          pltpu.VMEM((rpc,), jnp.int32),
          pltpu.VMEM((rpc,), jnp.int32),
          pltpu.VMEM((rpc, WIDE), jnp.float32),
          pltpu.VMEM((rpc, WIDE), jnp.float32),
          pltpu.VMEM((CB, HIST, HIDDEN), jnp.float32),
          pltpu.VMEM((CB, HIST, HIDDEN), jnp.float32),
          pltpu.SemaphoreType.DMA,
          pltpu.SemaphoreType.DMA,
          pltpu.SemaphoreType.DMA,
          pltpu.SemaphoreType.DMA,
          pltpu.SemaphoreType.DMA,
          pltpu.SemaphoreType.DMA,
      ],
  )
  def emb_kernel(x_hbm, packed_hbm, out_hbm, idx0, idx1, pidx0, pidx1,
                 gb0, gb1, ob0, ob1, gs0, gs1, os0, os1, is0, is1):
    idxs = (idx0, idx1)
    pidxs = (pidx0, pidx1)
    gbufs = (gb0, gb1)
    obufs = (ob0, ob1)
    g_sems = (gs0, gs1)
    o_sems = (os0, os1)
    i_sems = (is0, is1)

    wid = lax.axis_index("s") * NCORES + lax.axis_index("c")
    base = wid * items_per_w

    def idx_load(g, b):
      off = pl.multiple_of((base + g * CB) * HIST, 8)
      return pltpu.make_async_copy(
          x_hbm.at[pl.ds(off, rpc)], idxs[b], i_sems[b])

    def prep(b):
      idx_v = idxs[b]
      pidx = pidxs[b]

      def body(vi, carry):
        v = idx_v[pl.ds(vi * LANES, LANES)]
        pidx[pl.ds(vi * LANES, LANES)] = lax.shift_right_logical(v, 1)
        return carry

      lax.fori_loop(0, groups, body, 0)

    def gather(b):
      return pltpu.make_async_copy(
          packed_hbm.at[pidxs[b]], gbufs[b], g_sems[b])

    def put_one(g, b, bi):
      return pltpu.make_async_copy(
          obufs[b].at[bi], out_hbm.at[base + g * CB + bi], o_sems[b])

    def put_start(g, b):
      for bi in range(CB):
        put_one(g, b, bi).start()

    def put_wait(g, b):
      for bi in range(CB):
        put_one(g, b, bi).wait()

    def select_scale(b):
      idx_v = idxs[b]
      gbuf = gbufs[b]
      obuf = obufs[b]

      def body(vi, carry):
        v = idx_v[pl.ds(vi * LANES, LANES)]
        offs = lax.bitwise_and(v, 1) * HIDDEN
        for l in range(LANES):
          r = vi * LANES + l
          bi = r // HIST
          h = r - bi * HIST
          o = offs[l]
          for j in range(vpr):
            obuf[bi, h, pl.ds(j * LANES, LANES)] = (
                gbuf[r, pl.ds(o + j * LANES, LANES)] * SCALE)
        return carry

      lax.fori_loop(0, groups, body, 0)

    idx_load(0, 0).start()
    idx_load(0, 0).wait()
    prep(0)
    gather(0).start()
    idx_load(1, 1).start()

    def chunk_step(g, b):
      @pl.when(g + 1 < nchunks)
      def _():
        idx_load(g + 1, 1 - b).wait()
        prep(1 - b)
        gather(1 - b).start()

      gather(b).wait()

      @pl.when(g >= 2)
      def _():
        put_wait(g - 2, b)

      select_scale(b)

      @pl.when(g + 2 < nchunks)
      def _():
        idx_load(g + 2, b).start()

      put_start(g, b)

    def loop_body(g2, carry):
      for b in (0, 1):
        chunk_step(g2 * 2 + b, b)
      return carry

    lax.fori_loop(0, nchunks // 2, loop_body, 0)
    put_wait(nchunks - 2, 0)
    put_wait(nchunks - 1, 1)

  return emb_kernel


@jax.jit
def kernel(x, table):
  packed = _make_pack_kernel()(table)
  return _make_gather_kernel(x.shape[0])(x.reshape(-1), packed)


# speedup vs baseline: 1.8844x; 1.8844x over previous
import functools
import math

import jax
import jax.numpy as jnp
from jax import lax
from jax.experimental import pallas as pl
from jax.experimental.pallas import tpu as pltpu
from jax.experimental.pallas import tpu_sc as plsc

HIDDEN = 64
HIST = 20
SCALE = math.sqrt(HIDDEN)
LANES = 16
NCORES = 2
NSUB = 16
NW = NCORES * NSUB

CB = 16


def _make_kernel(batch: int):
  assert batch % (NW * CB) == 0
  items_per_w = batch // NW
  nchunks = items_per_w // CB
  rows_per_chunk = CB * HIST
  vpr = HIDDEN // LANES

  mesh = plsc.VectorSubcoreMesh(core_axis_name="c", subcore_axis_name="s")

  @functools.partial(
      pl.kernel,
      out_type=jax.ShapeDtypeStruct((batch, HIST, HIDDEN), jnp.float32),
      mesh=mesh,
      scratch_types=[
          pltpu.VMEM((CB * HIST,), jnp.int32),
          pltpu.VMEM((CB * HIST,), jnp.int32),
          pltpu.VMEM((CB, HIST, HIDDEN), jnp.float32),
          pltpu.VMEM((CB, HIST, HIDDEN), jnp.float32),
          pltpu.SemaphoreType.DMA,
          pltpu.SemaphoreType.DMA,
          pltpu.SemaphoreType.DMA,
          pltpu.SemaphoreType.DMA,
          pltpu.SemaphoreType.DMA,
          pltpu.SemaphoreType.DMA,
          pltpu.SemaphoreType.DMA,
          pltpu.SemaphoreType.DMA,
          pltpu.SemaphoreType.DMA,
          pltpu.SemaphoreType.DMA,
          pltpu.SemaphoreType.DMA,
          pltpu.SemaphoreType.DMA,
      ],
  )
  def emb_kernel(x_hbm, table_hbm, out_hbm, idx0, idx1, buf0, buf1,
                 gs00, gs01, gs02, gs03, gs10, gs11, gs12, gs13,
                 os0, os1, is0, is1):
    idxs = (idx0, idx1)
    bufs = (buf0, buf1)
    g_sems = ((gs00, gs01, gs02, gs03), (gs10, gs11, gs12, gs13))
    o_sems = (os0, os1)
    i_sems = (is0, is1)

    wid = lax.axis_index("s") * NCORES + lax.axis_index("c")
    base = wid * items_per_w

    def idx_load(g, b):
      return pltpu.make_async_copy(
          x_hbm.at[pl.ds((base + g * CB) * HIST, CB * HIST)], idxs[b],
          i_sems[b])

    def put(g, b):
      return pltpu.make_async_copy(
          bufs[b], out_hbm.at[pl.ds(base + g * CB, CB)], o_sems[b])

    def issue_gathers(b):
      buf = bufs[b]
      idx_v = idxs[b]

      def body(vi, carry):
        v = idx_v[pl.ds(vi * LANES, LANES)]
        for l in range(LANES):
          r = vi * LANES + l
          bi = r // HIST
          h = r - bi * HIST
          i = v[l]
          pltpu.make_async_copy(
              table_hbm.at[pl.ds(i, 1)], buf.at[bi, pl.ds(h, 1)],
              g_sems[b][l % 4]).start()
        return carry

      lax.fori_loop(0, rows_per_chunk // LANES, body, 0)

    def drain_gathers(g, b):
      qi = CB // 4
      for q in range(4):
        pltpu.make_async_copy(
            out_hbm.at[pl.ds(base + g * CB + q * qi, qi)],
            bufs[b].at[pl.ds(q * qi, qi)], g_sems[b][q]).wait()

    def scale(b):
      buf = bufs[b]

      @plsc.parallel_loop(0, rows_per_chunk * vpr, 1, unroll=8)
      def scale_body(i):
        r = i // vpr
        c = (i - r * vpr) * LANES
        bi = r // HIST
        h = r - bi * HIST
        buf[bi, h, pl.ds(c, LANES)] = buf[bi, h, pl.ds(c, LANES)] * SCALE

    idx_load(0, 0).start()
    idx_load(0, 0).wait()
    issue_gathers(0)
    if nchunks > 1:
      idx_load(1, 1).start()

    for g in range(nchunks):
      b = g % 2
      if g + 1 < nchunks:
        idx_load(g + 1, (g + 1) % 2).wait()
        if g >= 1:
          put(g - 1, 1 - b).wait()
        issue_gathers(1 - b)
        if g + 2 < nchunks:
          idx_load(g + 2, g % 2).start()
      drain_gathers(g, b)
      scale(b)
      put(g, b).start()

    if nchunks >= 2:
      put(nchunks - 2, nchunks % 2).wait()
    put(nchunks - 1, (nchunks - 1) % 2).wait()

  return emb_kernel


@jax.jit
def kernel(x, table):
  return _make_kernel(x.shape[0])(x.reshape(-1), table)
